# baseline (speedup 1.0000x reference)
import functools

import jax
import jax.numpy as jnp
from jax import lax
from jax.experimental import pallas as pl
from jax.experimental.pallas import tpu as pltpu
from jax.experimental.pallas import tpu_sc as plsc

N = 10000
DIM = 256

TILES = 16
NW = 2 * TILES
B = 128
BATCHES = 80
E_PAD = TILES * BATCHES * B
EPT = BATCHES * B
CAP = 11264
N_PAD = 10240
BAND = N_PAD // NW
GB = 128
WIN = 2048
CNT_LEN = NW * TILES * 16


@functools.cache
def _make_partition_sc():
    mesh = plsc.VectorSubcoreMesh(core_axis_name="c", subcore_axis_name="s")

    @functools.partial(
        pl.kernel,
        mesh=mesh,
        compiler_params=pltpu.CompilerParams(needs_layout_passes=False),
        out_type=(
            jax.ShapeDtypeStruct((NW * TILES * CAP,), jnp.int32),
            jax.ShapeDtypeStruct((NW * TILES * CAP,), jnp.int32),
            jax.ShapeDtypeStruct((NW * TILES * CAP,), jnp.float32),
            jax.ShapeDtypeStruct((CNT_LEN,), jnp.int32),
        ),
        scratch_types=[
            pltpu.VMEM((BATCHES, B), jnp.int32),
            pltpu.VMEM((BATCHES, B), jnp.int32),
            pltpu.VMEM((BATCHES, B), jnp.float32),
            pltpu.VMEM((CAP,), jnp.int32),
            pltpu.VMEM((CAP,), jnp.int32),
            pltpu.VMEM((CAP,), jnp.float32),
            pltpu.VMEM((16,), jnp.int32),
        ],
    )
    def _partition_sc(src_hbm, dst_hbm, w_hbm,
                      osrc, odst, ow, ocnt,
                      src_v, dst_v, w_v, bsrc, bdst, bw, cntv):
        c = lax.axis_index("c")
        s = lax.axis_index("s")

        pltpu.sync_copy(src_hbm.at[pl.ds(s * BATCHES, BATCHES)], src_v)
        pltpu.sync_copy(dst_hbm.at[pl.ds(s * BATCHES, BATCHES)], dst_v)
        pltpu.sync_copy(w_hbm.at[pl.ds(s * BATCHES, BATCHES)], w_v)

        zi = jnp.zeros((16,), jnp.int32)
        zf = jnp.zeros((16,), jnp.float32)

        def band_pass(bi, carry):
            band = c * TILES + bi
            lo = band * BAND

            def prefill(i, carry2):
                sl = pl.ds(i * 16, 16)
                bsrc[sl] = zi
                bdst[sl] = zi
                bw[sl] = zf
                return carry2

            lax.fori_loop(0, CAP // 16, prefill, 0)

            def chunk(i, off):
                r = i // (B // 16)
                col = (i % (B // 16)) * 16
                dv = dst_v[r, pl.ds(col, 16)]
                sv = src_v[r, pl.ds(col, 16)]
                wv = w_v[r, pl.ds(col, 16)]
                m = (dv >= lo) & (dv < lo + BAND)
                incl = plsc.cumsum(m.astype(jnp.int32))
                pos = off + incl - 1
                plsc.store_scatter(bdst, [pos], dv - lo, mask=m)
                plsc.store_scatter(bsrc, [pos], sv, mask=m)
                plsc.store_scatter(bw, [pos], wv, mask=m)
                return off + incl[15]

            cnt = lax.fori_loop(0, BATCHES * (B // 16), chunk, 0)

            base = (band * TILES + s) * CAP
            pltpu.sync_copy(bsrc, osrc.at[pl.ds(base, CAP)])
            pltpu.sync_copy(bdst, odst.at[pl.ds(base, CAP)])
            pltpu.sync_copy(bw, ow.at[pl.ds(base, CAP)])
            cntv[pl.ds(0, 16)] = jnp.full((16,), cnt, jnp.int32)
            pltpu.sync_copy(cntv, ocnt.at[pl.ds((band * TILES + s) * 16, 16)])
            return carry

        lax.fori_loop(0, TILES, band_pass, 0)

    return _partition_sc


@functools.cache
def _make_spmm_sc():
    mesh = plsc.VectorSubcoreMesh(core_axis_name="c", subcore_axis_name="s")

    @functools.partial(
        pl.kernel,
        mesh=mesh,
        compiler_params=pltpu.CompilerParams(needs_layout_passes=False),
        out_type=jax.ShapeDtypeStruct((N_PAD * DIM,), jnp.float32),
        scratch_types=[
            pltpu.VMEM((WIN // B, B), jnp.int32),
            pltpu.VMEM((WIN // B, B), jnp.int32),
            pltpu.VMEM((WIN // B, B), jnp.float32),
            pltpu.VMEM((256,), jnp.int32),
            pltpu.VMEM((GB, DIM), jnp.float32),
            pltpu.VMEM((BAND * DIM,), jnp.float32),
            pltpu.SemaphoreType.DMA,
        ],
    )
    def _spmm_sc(h_hbm, srcb, dstb, wbb, cnts, zeros_hbm, out_hbm,
                 src_v, dst_v, w_v, cnt_v, gbuf, acc, sem):
        c = lax.axis_index("c")
        s = lax.axis_index("s")
        w = c * TILES + s

        pltpu.sync_copy(cnts.at[pl.ds(w * TILES * 16, 256)], cnt_v)

        zf16 = jnp.zeros((16,), jnp.float32)

        def zero_body(i, carry):
            acc[pl.ds(i * 16, 16)] = zf16
            return carry

        lax.fori_loop(0, BAND * DIM // 16, zero_body, 0)

        def process(jb):
            iot = lax.iota(jnp.int32, 16)

            def group(g, carry):
                rvec = dst_v[jb, pl.ds(g * 16, 16)] * DIM
                wvec = w_v[jb, pl.ds(g * 16, 16)]
                for l in range(16):
                    rbase = jnp.full((16,), rvec[l], jnp.int32) + iot
                    wsc = wvec[l]
                    for k in range(DIM // 16):
                        plsc.addupdate_scatter(
                            acc, [rbase + (k * 16)],
                            gbuf[g * 16 + l, pl.ds(k * 16, 16)] * wsc)
                return carry

            lax.fori_loop(0, GB // 16, group, 0)

        def slice_body(t, carry):
            cvec = cnt_v[pl.ds(t * 16, 16)]
            cnt = cvec[0]
            base = (w * TILES + t) * (CAP // B)

            def win_body(v, carry2):
                wrow = v * (WIN // B)
                pltpu.sync_copy(srcb.at[pl.ds(base + wrow, WIN // B)], src_v)
                pltpu.sync_copy(dstb.at[pl.ds(base + wrow, WIN // B)], dst_v)
                pltpu.sync_copy(wbb.at[pl.ds(base + wrow, WIN // B)], w_v)
                rem = cnt - v * WIN
                nbv = lax.min(lax.div(rem + (GB - 1), GB), WIN // GB)

                def batch_body(jb, carry3):
                    pltpu.async_copy(h_hbm.at[src_v.at[jb]], gbuf, sem).wait()
                    process(jb)
                    return carry3

                lax.fori_loop(0, nbv, batch_body, 0)
                return carry2

            lax.fori_loop(0, lax.div(cnt + (WIN - 1), WIN), win_body, 0)
            return carry

        lax.fori_loop(0, TILES, slice_body, 0)

        pltpu.sync_copy(acc, out_hbm.at[pl.ds(w * (BAND * DIM), BAND * DIM)])

    return _spmm_sc


R = 1000
GRID = N // R


def _gru_body(x_ref, h1_ref, h2_ref, h3_ref, wih_ref, whh_ref,
              bih_ref, bhh_ref, out_ref):
    wih = wih_ref[...]
    whh = whh_ref[...]
    bih = bih_ref[...]
    bhh = bhh_ref[...]

    def step(xt, h):
        gi = jnp.dot(xt, wih, preferred_element_type=jnp.float32) + bih
        gh = jnp.dot(h, whh, preferred_element_type=jnp.float32) + bhh
        r = jax.nn.sigmoid(gi[:, :DIM] + gh[:, :DIM])
        z = jax.nn.sigmoid(gi[:, DIM:2 * DIM] + gh[:, DIM:2 * DIM])
        n = jnp.tanh(gi[:, 2 * DIM:] + r * gh[:, 2 * DIM:])
        return (1.0 - z) * n + z * h

    gi = jnp.dot(x_ref[...], wih, preferred_element_type=jnp.float32) + bih
    r = jax.nn.sigmoid(gi[:, :DIM] + bhh[:, :DIM])
    z = jax.nn.sigmoid(gi[:, DIM:2 * DIM] + bhh[:, DIM:2 * DIM])
    n = jnp.tanh(gi[:, 2 * DIM:] + r * bhh[:, 2 * DIM:])
    h = (1.0 - z) * n

    for ref in (h1_ref, h2_ref, h3_ref):
        h = step(ref[...], h)
    out_ref[...] = h


_gru_call = pl.pallas_call(
    _gru_body,
    grid=(GRID,),
    in_specs=[
        pl.BlockSpec((R, DIM), lambda i: (i, 0)),
        pl.BlockSpec((R, DIM), lambda i: (i, 0)),
        pl.BlockSpec((R, DIM), lambda i: (i, 0)),
        pl.BlockSpec((R, DIM), lambda i: (i, 0)),
        pl.BlockSpec((DIM, 3 * DIM), lambda i: (0, 0)),
        pl.BlockSpec((DIM, 3 * DIM), lambda i: (0, 0)),
        pl.BlockSpec((1, 3 * DIM), lambda i: (0, 0)),
        pl.BlockSpec((1, 3 * DIM), lambda i: (0, 0)),
    ],
    out_specs=pl.BlockSpec((R, DIM), lambda i: (i, 0)),
    out_shape=jax.ShapeDtypeStruct((N, DIM), jnp.float32),
)


def kernel(x, edge_index, edge_weight, W_ih, W_hh, b_ih, b_hh):
    dst = edge_index[0]
    src = edge_index[1]
    e = edge_weight.shape[0]
    pad = E_PAD - e
    src_p = jnp.concatenate([src, jnp.zeros((pad,), jnp.int32)])
    dst_p = jnp.concatenate([dst, jnp.arange(pad, dtype=jnp.int32) % N_PAD])
    w_p = jnp.concatenate([edge_weight, jnp.zeros((pad,), jnp.float32)])
    src_t = src_p.reshape(TILES * BATCHES, B)
    dst_t = dst_p.reshape(TILES * BATCHES, B)
    w_t = w_p.reshape(TILES * BATCHES, B)
    zeros = jnp.zeros((BAND * DIM,), jnp.float32)

    bsrc, bdst, bw, cnts = _make_partition_sc()(src_t, dst_t, w_t)
    bsrc = bsrc.reshape(NW * TILES * CAP // B, B)
    bdst = bdst.reshape(NW * TILES * CAP // B, B)
    bw = bw.reshape(NW * TILES * CAP // B, B)

    xp = jnp.concatenate([x, jnp.zeros((N_PAD - N, DIM), jnp.float32)], axis=0)

    spmm = _make_spmm_sc()
    h1 = spmm(xp, bsrc, bdst, bw, cnts, zeros).reshape(N_PAD, DIM)
    h2 = spmm(h1, bsrc, bdst, bw, cnts, zeros).reshape(N_PAD, DIM)
    h3 = spmm(h2, bsrc, bdst, bw, cnts, zeros).reshape(N_PAD, DIM)

    return _gru_call(
        x, h1, h2, h3,
        W_ih.T, W_hh.T,
        b_ih.reshape(1, 3 * DIM), b_hh.reshape(1, 3 * DIM),
    )


# speedup vs baseline: 1.4394x; 1.4394x over previous
import functools

import jax
import jax.numpy as jnp
from jax import lax
from jax.experimental import pallas as pl
from jax.experimental.pallas import tpu as pltpu
from jax.experimental.pallas import tpu_sc as plsc

N = 10000
DIM = 256

TILES = 16
NW = 2 * TILES
B = 128
BATCHES = 80
E_PAD = TILES * BATCHES * B
EPT = BATCHES * B
CAP = 11264
N_PAD = 10240
BAND = N_PAD // NW
GB = 64
WIN = 2048
PK = 16
CNT_LEN = NW * TILES * 16


@functools.cache
def _make_partition_sc():
    mesh = plsc.VectorSubcoreMesh(core_axis_name="c", subcore_axis_name="s")

    @functools.partial(
        pl.kernel,
        mesh=mesh,
        compiler_params=pltpu.CompilerParams(needs_layout_passes=False),
        out_type=(
            jax.ShapeDtypeStruct((NW * TILES * CAP,), jnp.int32),
            jax.ShapeDtypeStruct((NW * TILES * CAP,), jnp.int32),
            jax.ShapeDtypeStruct((NW * TILES * CAP,), jnp.float32),
            jax.ShapeDtypeStruct((CNT_LEN,), jnp.int32),
        ),
        scratch_types=[
            pltpu.VMEM((BATCHES, B), jnp.int32),
            pltpu.VMEM((BATCHES, B), jnp.int32),
            pltpu.VMEM((BATCHES, B), jnp.float32),
            pltpu.VMEM((CAP,), jnp.int32),
            pltpu.VMEM((CAP,), jnp.int32),
            pltpu.VMEM((CAP,), jnp.float32),
            pltpu.VMEM((16,), jnp.int32),
        ],
    )
    def _partition_sc(src_hbm, dst_hbm, w_hbm,
                      osrc, odst, ow, ocnt,
                      src_v, dst_v, w_v, bsrc, bdst, bw, cntv):
        c = lax.axis_index("c")
        s = lax.axis_index("s")

        pltpu.sync_copy(src_hbm.at[pl.ds(s * BATCHES, BATCHES)], src_v)
        pltpu.sync_copy(dst_hbm.at[pl.ds(s * BATCHES, BATCHES)], dst_v)
        pltpu.sync_copy(w_hbm.at[pl.ds(s * BATCHES, BATCHES)], w_v)

        zi = jnp.zeros((16,), jnp.int32)
        zf = jnp.zeros((16,), jnp.float32)

        def band_pass(bi, carry):
            band = c * TILES + bi
            lo = band * BAND

            def prefill(i, carry2):
                sl = pl.ds(i * 16, 16)
                bsrc[sl] = zi
                bdst[sl] = zi
                bw[sl] = zf
                return carry2

            lax.fori_loop(0, CAP // 16, prefill, 0)

            def chunk(i, off):
                r = i // (B // 16)
                col = (i % (B // 16)) * 16
                dv = dst_v[r, pl.ds(col, 16)]
                sv = src_v[r, pl.ds(col, 16)]
                wv = w_v[r, pl.ds(col, 16)]
                m = (dv >= lo) & (dv < lo + BAND)
                incl = plsc.cumsum(m.astype(jnp.int32))
                pos = off + incl - 1
                plsc.store_scatter(bdst, [pos], dv - lo, mask=m)
                plsc.store_scatter(bsrc, [pos], sv, mask=m)
                plsc.store_scatter(bw, [pos], wv, mask=m)
                return off + incl[15]

            cnt = lax.fori_loop(0, BATCHES * (B // 16), chunk, 0)

            base = (band * TILES + s) * CAP
            pltpu.sync_copy(bsrc, osrc.at[pl.ds(base, CAP)])
            pltpu.sync_copy(bdst, odst.at[pl.ds(base, CAP)])
            pltpu.sync_copy(bw, ow.at[pl.ds(base, CAP)])
            cntv[pl.ds(0, 16)] = jnp.full((16,), cnt, jnp.int32)
            pltpu.sync_copy(cntv, ocnt.at[pl.ds((band * TILES + s) * 16, 16)])
            return carry

        lax.fori_loop(0, TILES, band_pass, 0)

    return _partition_sc


@functools.cache
def _make_spmm_sc():
    mesh = plsc.VectorSubcoreMesh(core_axis_name="c", subcore_axis_name="s")

    @functools.partial(
        pl.kernel,
        mesh=mesh,
        compiler_params=pltpu.CompilerParams(needs_layout_passes=False),
        out_type=jax.ShapeDtypeStruct((N_PAD * DIM // 2,), jnp.int32),
        scratch_types=[
            pltpu.VMEM((WIN + 16,), jnp.int32),
            pltpu.VMEM((WIN + 16,), jnp.int32),
            pltpu.VMEM((WIN + 16,), jnp.float32),
            pltpu.VMEM((256,), jnp.int32),
            pltpu.VMEM((GB, DIM // 2), jnp.int32),
            pltpu.VMEM((GB, DIM // 2), jnp.int32),
            pltpu.VMEM((BAND * DIM,), jnp.float32),
            pltpu.VMEM((PK * DIM // 2,), jnp.int32),
            pltpu.SemaphoreType.DMA,
            pltpu.SemaphoreType.DMA,
        ],
    )
    def _spmm_sc(h_hbm, srcb, dstb, wbb, cnts, out_hbm,
                 src_v, dst_v, w_v, cnt_v, gbuf0, gbuf1, acc, pbuf,
                 sem0, sem1):
        c = lax.axis_index("c")
        s = lax.axis_index("s")
        w = c * TILES + s

        pltpu.sync_copy(cnts.at[pl.ds(w * TILES * 16, 256)], cnt_v)

        zf16 = jnp.zeros((16,), jnp.float32)

        def zero_body(i, carry):
            acc[pl.ds(i * 16, 16)] = zf16
            return carry

        lax.fori_loop(0, BAND * DIM // 16, zero_body, 0)

        def issue(jb, gb, sem):
            pltpu.async_copy(h_hbm.at[src_v.at[pl.ds(jb * GB, GB)]], gb, sem)

        def wait(jb, gb, sem):
            pltpu.make_async_copy(
                h_hbm.at[src_v.at[pl.ds(jb * GB, GB)]], gb, sem).wait()

        def process(jb, gb):
            eb = jb * GB
            iot2 = lax.iota(jnp.int32, 16) * 2

            def group(g, carry):
                go = eb + g * 16
                rvec = dst_v[pl.ds(go, 16)] * DIM
                wvec = w_v[pl.ds(go, 16)]
                for l in range(16):
                    reven = jnp.full((16,), rvec[l], jnp.int32) + iot2
                    wsc = wvec[l]
                    for k in range(DIM // 32):
                        ab = plsc.bitcast(
                            gb[g * 16 + l, pl.ds(k * 16, 16)], jnp.bfloat16)
                        a, b = plsc.unpack(ab, format=plsc.PackFormat.INTERLEAVED)
                        plsc.addupdate_scatter(
                            acc, [reven + (k * 32)], a * wsc)
                        plsc.addupdate_scatter(
                            acc, [reven + (k * 32 + 1)], b * wsc)
                return carry

            lax.fori_loop(0, GB // 16, group, 0)

        def slice_body(t, carry):
            cvec = cnt_v[pl.ds(t * 16, 16)]
            cnt = cvec[0]
            base = (w * TILES + t) * CAP

            def win_body(v, carry2):
                woff = v * WIN
                pltpu.sync_copy(srcb.at[pl.ds(base + woff, WIN)],
                                src_v.at[pl.ds(0, WIN)])
                pltpu.sync_copy(dstb.at[pl.ds(base + woff, WIN)],
                                dst_v.at[pl.ds(0, WIN)])
                pltpu.sync_copy(wbb.at[pl.ds(base + woff, WIN)],
                                w_v.at[pl.ds(0, WIN)])
                rem = cnt - woff
                nbv = lax.min(lax.div(rem + (GB - 1), GB), WIN // GB)

                @pl.when(nbv > 0)
                def _():
                    issue(0, gbuf0, sem0)

                def pair_body(i, carry3):
                    b0 = 2 * i
                    wait(b0, gbuf0, sem0)
                    @pl.when(b0 + 1 < nbv)
                    def _():
                        issue(b0 + 1, gbuf1, sem1)
                    process(b0, gbuf0)
                    @pl.when(b0 + 1 < nbv)
                    def _():
                        wait(b0 + 1, gbuf1, sem1)
                        @pl.when(b0 + 2 < nbv)
                        def _():
                            issue(b0 + 2, gbuf0, sem0)
                        process(b0 + 1, gbuf1)
                    return carry3

                lax.fori_loop(0, lax.div(nbv + 1, 2), pair_body, 0)
                return carry2

            lax.fori_loop(0, lax.div(cnt + (WIN - 1), WIN), win_body, 0)
            return carry

        lax.fori_loop(0, TILES, slice_body, 0)

        def wb_body(ch, carry):
            fbase = ch * (PK * DIM)
            iot2w = lax.iota(jnp.int32, 16) * 2

            def pk_body(g, carry2):
                gbase = jnp.full((16,), fbase + g * 32, jnp.int32) + iot2w
                a = plsc.load_gather(acc, [gbase])
                b = plsc.load_gather(acc, [gbase + 1])
                pbuf[pl.ds(g * 16, 16)] = plsc.bitcast(
                    plsc.pack(a, b, format=plsc.PackFormat.INTERLEAVED),
                    jnp.int32)
                return carry2

            lax.fori_loop(0, PK * DIM // 32, pk_body, 0)
            pltpu.sync_copy(
                pbuf,
                out_hbm.at[pl.ds(w * (BAND * DIM // 2) + ch * (PK * DIM // 2),
                                 PK * DIM // 2)])
            return carry

        lax.fori_loop(0, BAND // PK, wb_body, 0)

    return _spmm_sc


R = 1000
GRID = N // R


def _gru_body(x_ref, h1_ref, h2_ref, h3_ref, wih_ref, whh_ref,
              bih_ref, bhh_ref, out_ref):
    wih = wih_ref[...]
    whh = whh_ref[...]
    bih = bih_ref[...]
    bhh = bhh_ref[...]

    def step(xt, h):
        gi = jnp.dot(xt, wih, preferred_element_type=jnp.float32) + bih
        gh = jnp.dot(h, whh, preferred_element_type=jnp.float32) + bhh
        r = jax.nn.sigmoid(gi[:, :DIM] + gh[:, :DIM])
        z = jax.nn.sigmoid(gi[:, DIM:2 * DIM] + gh[:, DIM:2 * DIM])
        n = jnp.tanh(gi[:, 2 * DIM:] + r * gh[:, 2 * DIM:])
        return (1.0 - z) * n + z * h

    gi = jnp.dot(x_ref[...], wih, preferred_element_type=jnp.float32) + bih
    r = jax.nn.sigmoid(gi[:, :DIM] + bhh[:, :DIM])
    z = jax.nn.sigmoid(gi[:, DIM:2 * DIM] + bhh[:, DIM:2 * DIM])
    n = jnp.tanh(gi[:, 2 * DIM:] + r * bhh[:, 2 * DIM:])
    h = (1.0 - z) * n

    for ref in (h1_ref, h2_ref, h3_ref):
        h = step(ref[...], h)
    out_ref[...] = h


_gru_call = pl.pallas_call(
    _gru_body,
    grid=(GRID,),
    in_specs=[
        pl.BlockSpec((R, DIM), lambda i: (i, 0)),
        pl.BlockSpec((R, DIM), lambda i: (i, 0)),
        pl.BlockSpec((R, DIM), lambda i: (i, 0)),
        pl.BlockSpec((R, DIM), lambda i: (i, 0)),
        pl.BlockSpec((DIM, 3 * DIM), lambda i: (0, 0)),
        pl.BlockSpec((DIM, 3 * DIM), lambda i: (0, 0)),
        pl.BlockSpec((1, 3 * DIM), lambda i: (0, 0)),
        pl.BlockSpec((1, 3 * DIM), lambda i: (0, 0)),
    ],
    out_specs=pl.BlockSpec((R, DIM), lambda i: (i, 0)),
    out_shape=jax.ShapeDtypeStruct((N, DIM), jnp.float32),
)


def kernel(x, edge_index, edge_weight, W_ih, W_hh, b_ih, b_hh):
    dst = edge_index[0]
    src = edge_index[1]
    e = edge_weight.shape[0]
    pad = E_PAD - e
    src_p = jnp.concatenate([src, jnp.zeros((pad,), jnp.int32)])
    dst_p = jnp.concatenate([dst, jnp.arange(pad, dtype=jnp.int32) % N_PAD])
    w_p = jnp.concatenate([edge_weight, jnp.zeros((pad,), jnp.float32)])
    src_t = src_p.reshape(TILES * BATCHES, B)
    dst_t = dst_p.reshape(TILES * BATCHES, B)
    w_t = w_p.reshape(TILES * BATCHES, B)

    bsrc, bdst, bw, cnts = _make_partition_sc()(src_t, dst_t, w_t)

    xp = jnp.concatenate(
        [x, jnp.zeros((N_PAD - N, DIM), jnp.float32)], axis=0
    ).astype(jnp.bfloat16)
    xpi = lax.bitcast_convert_type(
        xp.reshape(N_PAD, DIM // 2, 2), jnp.int32)

    def as_bf16(hflat):
        return lax.bitcast_convert_type(
            hflat.reshape(N_PAD, DIM // 2), jnp.bfloat16).reshape(N_PAD, DIM)

    spmm = _make_spmm_sc()
    h1 = spmm(xpi, bsrc, bdst, bw, cnts)
    h2 = spmm(h1.reshape(N_PAD, DIM // 2), bsrc, bdst, bw, cnts)
    h3 = spmm(h2.reshape(N_PAD, DIM // 2), bsrc, bdst, bw, cnts)

    return _gru_call(
        x, as_bf16(h1), as_bf16(h2), as_bf16(h3),
        W_ih.T, W_hh.T,
        b_ih.reshape(1, 3 * DIM), b_hh.reshape(1, 3 * DIM),
    )
